# baseline (speedup 1.0000x reference)
import jax
import jax.numpy as jnp
from jax.experimental import pallas as pl
from jax.experimental.pallas import tpu as pltpu


def _lenet_kernel(x_ref, ipm_ref, wc1_ref, bc1_ref, wc2_ref, bc2_ref,
                  w1_ref, b1_ref, w2_ref, b2_ref, w3_ref, b3_ref, o_ref):
    xb = x_ref[...]
    ip = ipm_ref[...]
    bc1 = bc1_ref[...][:, :, None]
    bc2 = bc2_ref[...][:, :, None]

    tbs = [jnp.einsum("mk,nk->mn", ip, xb[:, b * 128:(b + 1) * 128],
                      preferred_element_type=jnp.float32).astype(jnp.bfloat16)
           for b in range(24)]
    pc = [[jnp.concatenate([tbs[c * 8 + hh][q * 32:(q + 1) * 32, :]
                            for hh in range(8)])
           for c in range(3)] for q in range(4)]
    ps = [[jnp.pad(p[1:, :], ((0, 1), (0, 0))) for p in row] for row in pc]
    planes = pc + ps

    taps = [planes[t][c][i * 8:i * 8 + 224, :]
            for c in range(3) for i in range(5) for t in range(8)]
    p = jnp.stack(taps)
    y = jnp.einsum("qk,ksm->qsm", wc1_ref[...], p,
                   preferred_element_type=jnp.float32)
    y = jnp.maximum(y + bc1, 0.0)
    y = y.reshape(2, 2, 6, 224, -1)
    y = jnp.maximum(y[:, 0], y[:, 1])
    y = y.reshape(2, 6, 14, 2, 8, -1)
    y = jnp.maximum(y[:, :, :, 0], y[:, :, :, 1])
    h1 = y.reshape(2, 6, 112, -1).astype(jnp.bfloat16)
    h1 = jnp.pad(h1, ((0, 0), (0, 0), (0, 8), (0, 0)))
    h1s = [h1,
           jnp.pad(h1[:, :, 1:, :], ((0, 0), (0, 0), (0, 1), (0, 0))),
           jnp.pad(h1[:, :, 2:, :], ((0, 0), (0, 0), (0, 2), (0, 0)))]

    taps = [h1s[u // 2][u % 2, c, i * 8:i * 8 + 80, :]
            for c in range(6) for i in range(5) for u in range(6)]
    p = jnp.stack(taps)
    y = jnp.einsum("qk,ksm->qsm", wc2_ref[...], p,
                   preferred_element_type=jnp.float32)
    y = jnp.maximum(y + bc2, 0.0)
    y = y.reshape(2, 16, 80, -1)
    y = jnp.maximum(y[0], y[1])
    y = y.reshape(16, 5, 2, 8, -1)
    h2 = jnp.maximum(y[:, :, 0], y[:, :, 1])
    h2 = h2.reshape(16, 40, -1).reshape(640, -1)

    h = jnp.dot(w1_ref[...], h2, preferred_element_type=jnp.float32)
    h = jnp.maximum(h + b1_ref[...], 0.0)
    h = jnp.dot(w2_ref[...], h, preferred_element_type=jnp.float32)
    h = jnp.maximum(h + b2_ref[...], 0.0)
    o_ref[...] = (jnp.dot(w3_ref[...], h,
                          preferred_element_type=jnp.float32)
                  + b3_ref[...])


def kernel(c1_w, c1_b, c2_w, c2_b, w1, b1, w2, b2, w3, b3, x):
    B = x.shape[0]
    TB = 128

    xb = x.reshape(B, 3072).astype(jnp.bfloat16)
    n = jnp.arange(128)
    m = (n % 4) * 32 + (n // 32) * 8 + (n % 32) // 4
    ipm = (jnp.arange(128)[:, None] == m[None, :]).astype(jnp.bfloat16)

    wr = c1_w.reshape(6, 3, 5, 5)
    wc1 = jnp.stack([jnp.pad(wr, ((0, 0), (0, 0), (0, 0), (d, 3 - d)))
                     for d in range(4)])
    wc1 = wc1.reshape(4, 6, 120).reshape(24, 120).astype(jnp.bfloat16)
    bc1 = jnp.concatenate([c1_b] * 4, axis=0)

    w2r = c2_w.reshape(16, 6, 5, 5)
    wc2 = jnp.stack([jnp.pad(w2r, ((0, 0), (0, 0), (0, 0), (e2, 1 - e2)))
                     for e2 in range(2)])
    wc2 = wc2.reshape(2, 16, 180).reshape(32, 180).astype(jnp.bfloat16)
    bc2 = jnp.concatenate([c2_b] * 2, axis=0)

    w1t = w1.T.reshape(128, 16, 5, 5)
    w1t = jnp.pad(w1t, ((0, 0), (0, 0), (0, 0), (0, 3)))
    w1t = w1t.reshape(128, 640)

    out = pl.pallas_call(
        _lenet_kernel,
        out_shape=jax.ShapeDtypeStruct((10, B), jnp.float32),
        grid=(B // TB,),
        in_specs=[
            pl.BlockSpec((TB, 3072), lambda t: (t, 0)),
            pl.BlockSpec((128, 128), lambda t: (0, 0)),
            pl.BlockSpec((24, 120), lambda t: (0, 0)),
            pl.BlockSpec((24, 1), lambda t: (0, 0)),
            pl.BlockSpec((32, 180), lambda t: (0, 0)),
            pl.BlockSpec((32, 1), lambda t: (0, 0)),
            pl.BlockSpec((128, 640), lambda t: (0, 0)),
            pl.BlockSpec((128, 1), lambda t: (0, 0)),
            pl.BlockSpec((128, 128), lambda t: (0, 0)),
            pl.BlockSpec((128, 1), lambda t: (0, 0)),
            pl.BlockSpec((10, 128), lambda t: (0, 0)),
            pl.BlockSpec((10, 1), lambda t: (0, 0)),
        ],
        out_specs=pl.BlockSpec((10, TB), lambda t: (0, t)),
        compiler_params=pltpu.CompilerParams(
            dimension_semantics=("parallel",)),
        cost_estimate=pl.CostEstimate(
            flops=2 * B * (120 * 24 * 224 + 180 * 32 * 80
                           + 640 * 128 + 128 * 128 + 128 * 10),
            transcendentals=0,
            bytes_accessed=4 * (3072 * B + 10 * B)),
    )(xb, ipm, wc1, bc1, wc2, bc2,
      w1t, b1.T, w2.T, b2.T, w3.T, b3.T)
    return out.T


# speedup vs baseline: 1.0959x; 1.0959x over previous
import jax
import jax.numpy as jnp
from jax.experimental import pallas as pl
from jax.experimental.pallas import tpu as pltpu


def _lenet_kernel(x_ref, ipm_ref, wc1_ref, bc1_ref, wc2_ref, bc2_ref,
                  w1_ref, b1_ref, w2_ref, b2_ref, w3_ref, b3_ref, o_ref):
    xb = x_ref[...]
    ip = ipm_ref[...]
    bc1 = bc1_ref[...][:, :, None]
    bc2 = bc2_ref[...][:, :, None]

    tbs = [jnp.einsum("mk,nk->mn", ip, xb[:, b * 128:(b + 1) * 128],
                      preferred_element_type=jnp.float32) for b in range(24)]
    pc = [[jnp.concatenate([tbs[c * 8 + hh][q * 32:(q + 1) * 32, :]
                            for hh in range(8)])
           for c in range(3)] for q in range(4)]
    ps = [[jnp.pad(p[1:, :], ((0, 1), (0, 0))) for p in row] for row in pc]
    planes = pc + ps

    taps = [planes[t][c][i * 8:i * 8 + 224, :]
            for c in range(3) for i in range(5) for t in range(8)]
    p = jnp.stack(taps)
    y = jnp.einsum("qk,ksm->qsm", wc1_ref[...], p,
                   preferred_element_type=jnp.float32)
    y = jnp.maximum(y + bc1, 0.0)
    y = y.reshape(2, 2, 6, 224, -1)
    y = jnp.maximum(y[:, 0], y[:, 1])
    y = y.reshape(2, 6, 14, 2, 8, -1)
    y = jnp.maximum(y[:, :, :, 0], y[:, :, :, 1])
    h1 = y.reshape(2, 6, 112, -1)
    h1 = jnp.pad(h1, ((0, 0), (0, 0), (0, 8), (0, 0)))
    h1s = [h1,
           jnp.pad(h1[:, :, 1:, :], ((0, 0), (0, 0), (0, 1), (0, 0))),
           jnp.pad(h1[:, :, 2:, :], ((0, 0), (0, 0), (0, 2), (0, 0)))]

    taps = [h1s[u // 2][u % 2, c, i * 8:i * 8 + 80, :]
            for c in range(6) for i in range(5) for u in range(6)]
    p = jnp.stack(taps)
    y = jnp.einsum("qk,ksm->qsm", wc2_ref[...], p,
                   preferred_element_type=jnp.float32)
    y = jnp.maximum(y + bc2, 0.0)
    y = y.reshape(2, 16, 80, -1)
    y = jnp.maximum(y[0], y[1])
    y = y.reshape(16, 5, 2, 8, -1)
    h2 = jnp.maximum(y[:, :, 0], y[:, :, 1])
    h2 = h2.reshape(16, 40, -1).reshape(640, -1)

    h = jnp.dot(w1_ref[...], h2, preferred_element_type=jnp.float32)
    h = jnp.maximum(h + b1_ref[...], 0.0)
    h = jnp.dot(w2_ref[...], h, preferred_element_type=jnp.float32)
    h = jnp.maximum(h + b2_ref[...], 0.0)
    o_ref[...] = (jnp.dot(w3_ref[...], h,
                          preferred_element_type=jnp.float32)
                  + b3_ref[...])


def kernel(c1_w, c1_b, c2_w, c2_b, w1, b1, w2, b2, w3, b3, x):
    B = x.shape[0]
    TB = 256

    xb = x.reshape(B, 3072).astype(jnp.bfloat16)
    n = jnp.arange(128)
    m = (n % 4) * 32 + (n // 32) * 8 + (n % 32) // 4
    ipm = (jnp.arange(128)[:, None] == m[None, :]).astype(jnp.bfloat16)

    wr = c1_w.reshape(6, 3, 5, 5)
    wc1 = jnp.stack([jnp.pad(wr, ((0, 0), (0, 0), (0, 0), (d, 3 - d)))
                     for d in range(4)])
    wc1 = wc1.reshape(4, 6, 120).reshape(24, 120)
    bc1 = jnp.concatenate([c1_b] * 4, axis=0)

    w2r = c2_w.reshape(16, 6, 5, 5)
    wc2 = jnp.stack([jnp.pad(w2r, ((0, 0), (0, 0), (0, 0), (e2, 1 - e2)))
                     for e2 in range(2)])
    wc2 = wc2.reshape(2, 16, 180).reshape(32, 180)
    bc2 = jnp.concatenate([c2_b] * 2, axis=0)

    w1t = w1.T.reshape(128, 16, 5, 5)
    w1t = jnp.pad(w1t, ((0, 0), (0, 0), (0, 0), (0, 3)))
    w1t = w1t.reshape(128, 640)

    out = pl.pallas_call(
        _lenet_kernel,
        out_shape=jax.ShapeDtypeStruct((10, B), jnp.float32),
        grid=(B // TB,),
        in_specs=[
            pl.BlockSpec((TB, 3072), lambda t: (t, 0)),
            pl.BlockSpec((128, 128), lambda t: (0, 0)),
            pl.BlockSpec((24, 120), lambda t: (0, 0)),
            pl.BlockSpec((24, 1), lambda t: (0, 0)),
            pl.BlockSpec((32, 180), lambda t: (0, 0)),
            pl.BlockSpec((32, 1), lambda t: (0, 0)),
            pl.BlockSpec((128, 640), lambda t: (0, 0)),
            pl.BlockSpec((128, 1), lambda t: (0, 0)),
            pl.BlockSpec((128, 128), lambda t: (0, 0)),
            pl.BlockSpec((128, 1), lambda t: (0, 0)),
            pl.BlockSpec((10, 128), lambda t: (0, 0)),
            pl.BlockSpec((10, 1), lambda t: (0, 0)),
        ],
        out_specs=pl.BlockSpec((10, TB), lambda t: (0, t)),
        compiler_params=pltpu.CompilerParams(
            dimension_semantics=("parallel",)),
        cost_estimate=pl.CostEstimate(
            flops=2 * B * (120 * 24 * 224 + 180 * 32 * 80
                           + 640 * 128 + 128 * 128 + 128 * 10),
            transcendentals=0,
            bytes_accessed=4 * (3072 * B + 10 * B)),
    )(xb, ipm, wc1, bc1, wc2, bc2,
      w1t, b1.T, w2.T, b2.T, w3.T, b3.T)
    return out.T


# speedup vs baseline: 1.1033x; 1.0067x over previous
import jax
import jax.numpy as jnp
from jax.experimental import pallas as pl
from jax.experimental.pallas import tpu as pltpu


def _lenet_kernel(x_ref, ipm_ref, wc1_ref, bc1_ref, wc2_ref, bc2_ref,
                  w1_ref, b1_ref, w2_ref, b2_ref, w3_ref, b3_ref, o_ref):
    xb = x_ref[...]
    ip = ipm_ref[...]
    bc1 = bc1_ref[...][:, :, None]
    bc2 = bc2_ref[...][:, :, None]

    tbs = [jnp.einsum("mk,nk->mn", ip, xb[:, b * 128:(b + 1) * 128],
                      preferred_element_type=jnp.float32) for b in range(24)]
    pc = [[jnp.concatenate([tbs[c * 8 + hh][q * 32:(q + 1) * 32, :]
                            for hh in range(8)])
           for c in range(3)] for q in range(4)]
    ps = [[jnp.pad(p[1:, :], ((0, 1), (0, 0))) for p in row] for row in pc]
    planes = pc + ps

    taps = [planes[t][c][i * 8:i * 8 + 224, :]
            for c in range(3) for i in range(5) for t in range(8)]
    p = jnp.stack(taps)
    y = jnp.einsum("qk,ksm->qsm", wc1_ref[...], p,
                   preferred_element_type=jnp.float32)
    y = jnp.maximum(y + bc1, 0.0)
    y = y.reshape(2, 2, 6, 224, -1)
    y = jnp.maximum(y[:, 0], y[:, 1])
    y = y.reshape(2, 6, 14, 2, 8, -1)
    y = jnp.maximum(y[:, :, :, 0], y[:, :, :, 1])
    h1 = y.reshape(2, 6, 112, -1)
    h1 = jnp.pad(h1, ((0, 0), (0, 0), (0, 8), (0, 0)))
    h1s = [h1,
           jnp.pad(h1[:, :, 1:, :], ((0, 0), (0, 0), (0, 1), (0, 0))),
           jnp.pad(h1[:, :, 2:, :], ((0, 0), (0, 0), (0, 2), (0, 0)))]

    taps = [h1s[u // 2][u % 2, c, i * 8:i * 8 + 80, :]
            for c in range(6) for i in range(5) for u in range(6)]
    p = jnp.stack(taps)
    y = jnp.einsum("qk,ksm->qsm", wc2_ref[...], p,
                   preferred_element_type=jnp.float32)
    y = jnp.maximum(y + bc2, 0.0)
    y = y.reshape(2, 16, 80, -1)
    y = jnp.maximum(y[0], y[1])
    y = y.reshape(16, 5, 2, 8, -1)
    h2 = jnp.maximum(y[:, :, 0], y[:, :, 1])
    h2 = h2.reshape(16, 40, -1).reshape(640, -1)

    h = jnp.dot(w1_ref[...], h2, preferred_element_type=jnp.float32)
    h = jnp.maximum(h + b1_ref[...], 0.0)
    h = jnp.dot(w2_ref[...], h, preferred_element_type=jnp.float32)
    h = jnp.maximum(h + b2_ref[...], 0.0)
    o_ref[...] = (jnp.dot(w3_ref[...], h,
                          preferred_element_type=jnp.float32)
                  + b3_ref[...])


def kernel(c1_w, c1_b, c2_w, c2_b, w1, b1, w2, b2, w3, b3, x):
    B = x.shape[0]
    TB = 256

    xb = x.reshape(B, 3072)
    n = jnp.arange(128)
    m = (n % 4) * 32 + (n // 32) * 8 + (n % 32) // 4
    ipm = (jnp.arange(128)[:, None] == m[None, :]).astype(jnp.float32)

    wr = c1_w.reshape(6, 3, 5, 5)
    wc1 = jnp.stack([jnp.pad(wr, ((0, 0), (0, 0), (0, 0), (d, 3 - d)))
                     for d in range(4)])
    wc1 = wc1.reshape(4, 6, 120).reshape(24, 120)
    bc1 = jnp.concatenate([c1_b] * 4, axis=0)

    w2r = c2_w.reshape(16, 6, 5, 5)
    wc2 = jnp.stack([jnp.pad(w2r, ((0, 0), (0, 0), (0, 0), (e2, 1 - e2)))
                     for e2 in range(2)])
    wc2 = wc2.reshape(2, 16, 180).reshape(32, 180)
    bc2 = jnp.concatenate([c2_b] * 2, axis=0)

    w1t = w1.T.reshape(128, 16, 5, 5)
    w1t = jnp.pad(w1t, ((0, 0), (0, 0), (0, 0), (0, 3)))
    w1t = w1t.reshape(128, 640)

    out = pl.pallas_call(
        _lenet_kernel,
        out_shape=jax.ShapeDtypeStruct((10, B), jnp.float32),
        grid=(B // TB,),
        in_specs=[
            pl.BlockSpec((TB, 3072), lambda t: (t, 0)),
            pl.BlockSpec((128, 128), lambda t: (0, 0)),
            pl.BlockSpec((24, 120), lambda t: (0, 0)),
            pl.BlockSpec((24, 1), lambda t: (0, 0)),
            pl.BlockSpec((32, 180), lambda t: (0, 0)),
            pl.BlockSpec((32, 1), lambda t: (0, 0)),
            pl.BlockSpec((128, 640), lambda t: (0, 0)),
            pl.BlockSpec((128, 1), lambda t: (0, 0)),
            pl.BlockSpec((128, 128), lambda t: (0, 0)),
            pl.BlockSpec((128, 1), lambda t: (0, 0)),
            pl.BlockSpec((10, 128), lambda t: (0, 0)),
            pl.BlockSpec((10, 1), lambda t: (0, 0)),
        ],
        out_specs=pl.BlockSpec((10, TB), lambda t: (0, t)),
        compiler_params=pltpu.CompilerParams(
            dimension_semantics=("parallel",)),
        cost_estimate=pl.CostEstimate(
            flops=2 * B * (120 * 24 * 224 + 180 * 32 * 80
                           + 640 * 128 + 128 * 128 + 128 * 10),
            transcendentals=0,
            bytes_accessed=4 * (3072 * B + 10 * B)),
    )(xb, ipm, wc1, bc1, wc2, bc2,
      w1t, b1.T, w2.T, b2.T, w3.T, b3.T)
    return out.T
